# Initial kernel scaffold; baseline (speedup 1.0000x reference)
import jax
import jax.numpy as jnp
from jax.experimental import pallas as pl


def kernel(h_subgraph, subgraph_idx_batch, W1, b1, W2, b2, S1, bs1, S2, bs2, Fw1, Fb1, Fw2, Fb2):
    raise NotImplementedError("write your pallas kernel here")


# speedup vs baseline: 1.8689x; 1.8689x over previous
import functools

import jax
import jax.numpy as jnp
from jax import lax
from jax.experimental import pallas as pl
from jax.experimental.pallas import tpu as pltpu
from jax.experimental.pallas import tpu_sc as plsc

D = 128
G = 10000
GP = 10240
_NC = 2
_NS = 16
_NW = _NC * _NS
_CH = 80
_VPR = D // 16
_GS = GP // _NS

_F32 = jnp.float32


def _sc_mesh():
    return plsc.VectorSubcoreMesh(core_axis_name="c", subcore_axis_name="s")


def _zero_fill(ref, rows):
    z = jnp.zeros((16,), _F32)

    def body(i, c):
        for j in range(_VPR):
            ref[i, pl.ds(j * 16, 16)] = z
        return c

    lax.fori_loop(0, rows, body, 0)


def _fill_rows16(ref, rows, value):
    v = jnp.full((16,), value, _F32)

    def body(i, c):
        ref[i] = v
        return c

    lax.fori_loop(0, rows, body, 0)


def _make_sc_pool(n):
    rw = n // _NW
    nch = rw // _CH

    @functools.partial(
        pl.kernel,
        out_type=[
            jax.ShapeDtypeStruct((2 * GP, D), _F32),
        ],
        mesh=_sc_mesh(),
        scratch_types=[
            pltpu.VMEM((_CH, D), _F32),
            pltpu.VMEM((_CH,), jnp.int32),
            pltpu.VMEM((128, D), _F32),
            pltpu.VMEM_SHARED((GP, D), _F32),
            pltpu.SemaphoreType.DMA,
        ],
    )
    def sc_pool(h_hbm, ids_hbm, psum_hbm,
                hbuf, idbuf, zbuf, sum_sh, sem_s):
        cid = lax.axis_index("c")
        sid = lax.axis_index("s")
        wid = sid * _NC + cid

        _zero_fill(zbuf, 128)
        for k in range(_GS // 128):
            pltpu.sync_copy(zbuf, sum_sh.at[pl.ds(sid * _GS + k * 128, 128)])
        plsc.subcore_barrier()

        def chunk(c, carry):
            row0 = wid * rw + c * _CH
            pltpu.sync_copy(h_hbm.at[pl.ds(row0, _CH)], hbuf)
            pltpu.sync_copy(ids_hbm.at[pl.ds(row0, _CH)], idbuf)
            pltpu.async_copy(hbuf, sum_sh.at[idbuf], sem_s, add=True).wait()
            return carry

        lax.fori_loop(0, nch, chunk, 0)
        plsc.subcore_barrier()

        out0 = cid * GP + sid * _GS
        pltpu.sync_copy(sum_sh.at[pl.ds(sid * _GS, _GS)],
                        psum_hbm.at[pl.ds(out0, _GS)])

    return sc_pool


def _make_sc_count(n):
    rw = n // _NW
    nch = rw // _CH

    @functools.partial(
        pl.kernel,
        out_type=[jax.ShapeDtypeStruct((2 * GP, D), _F32)],
        mesh=_sc_mesh(),
        scratch_types=[
            pltpu.VMEM((_CH, D), _F32),
            pltpu.VMEM((_CH,), jnp.int32),
            pltpu.VMEM((128, D), _F32),
            pltpu.VMEM_SHARED((GP, D), _F32),
            pltpu.SemaphoreType.DMA,
        ],
    )
    def sc_count(ids_hbm, pcnt_hbm, ones_v, idbuf, zbuf, cnt_sh, sem_c):
        cid = lax.axis_index("c")
        sid = lax.axis_index("s")
        wid = sid * _NC + cid

        _zero_fill(zbuf, 128)
        one = jnp.full((16,), 1.0, _F32)

        def ofill(i, c):
            for j in range(_VPR):
                ones_v[i, pl.ds(j * 16, 16)] = one
            return c

        lax.fori_loop(0, _CH, ofill, 0)
        for k in range(_GS // 128):
            pltpu.sync_copy(zbuf, cnt_sh.at[pl.ds(sid * _GS + k * 128, 128)])
        plsc.subcore_barrier()

        def chunk(c, carry):
            row0 = wid * rw + c * _CH
            pltpu.sync_copy(ids_hbm.at[pl.ds(row0, _CH)], idbuf)
            pltpu.async_copy(ones_v, cnt_sh.at[idbuf], sem_c, add=True).wait()
            return carry

        lax.fori_loop(0, nch, chunk, 0)
        plsc.subcore_barrier()

        out0 = cid * GP + sid * _GS
        pltpu.sync_copy(cnt_sh.at[pl.ds(sid * _GS, _GS)],
                        pcnt_hbm.at[pl.ds(out0, _GS)])

    return sc_count


def _make_sc_update(n, write_rows):
    rw = n // _NW
    nch = rw // _CH

    out_type = [jax.ShapeDtypeStruct((2 * GP, D), _F32)]
    if write_rows:
        out_type = [jax.ShapeDtypeStruct((n, D), _F32)] + out_type

    @functools.partial(
        pl.kernel,
        out_type=out_type,
        mesh=_sc_mesh(),
        scratch_types=[
            pltpu.VMEM((_CH, D), _F32),
            pltpu.VMEM((_CH, D), _F32),
            pltpu.VMEM((_CH, D), _F32),
            pltpu.VMEM((_CH,), jnp.int32),
            pltpu.VMEM((128, D), _F32),
            pltpu.VMEM_SHARED((GP, D), _F32),
            pltpu.SemaphoreType.DMA,
            pltpu.SemaphoreType.DMA,
        ],
    )
    def sc_update(x1_hbm, x2_hbm, ids_hbm, *rest):
        if write_rows:
            (hout_hbm, psum_hbm,
             xbuf, gbuf, obuf, idbuf, zbuf, sum_sh, sem_g, sem_s) = rest
        else:
            (psum_hbm,
             xbuf, gbuf, obuf, idbuf, zbuf, sum_sh, sem_g, sem_s) = rest
        cid = lax.axis_index("c")
        sid = lax.axis_index("s")
        wid = sid * _NC + cid

        _zero_fill(zbuf, 128)
        for k in range(_GS // 128):
            pltpu.sync_copy(zbuf, sum_sh.at[pl.ds(sid * _GS + k * 128, 128)])
        plsc.subcore_barrier()

        def chunk(c, carry):
            row0 = wid * rw + c * _CH
            pltpu.sync_copy(x1_hbm.at[pl.ds(row0, _CH)], xbuf)
            pltpu.sync_copy(ids_hbm.at[pl.ds(row0, _CH)], idbuf)
            pltpu.async_copy(x2_hbm.at[idbuf], gbuf, sem_g).wait()

            def row(r, cc):
                for j in range(_VPR):
                    sl = pl.ds(j * 16, 16)
                    v = xbuf[r, sl] + gbuf[r, sl]
                    obuf[r, sl] = jnp.where(v > 0.0, v, jnp.exp(v) - 1.0)
                return cc

            lax.fori_loop(0, _CH, row, 0)
            if write_rows:
                pltpu.sync_copy(obuf, hout_hbm.at[pl.ds(row0, _CH)])
            pltpu.async_copy(obuf, sum_sh.at[idbuf], sem_s, add=True).wait()
            return carry

        lax.fori_loop(0, nch, chunk, 0)
        plsc.subcore_barrier()

        out0 = cid * GP + sid * _GS
        pltpu.sync_copy(sum_sh.at[pl.ds(sid * _GS, _GS)],
                        psum_hbm.at[pl.ds(out0, _GS)])

    return sc_update


def _mm_bias(h, w, b, br=3200):
    n = h.shape[0]

    def body(h_ref, w_ref, b_ref, o_ref):
        o_ref[...] = (
            jnp.dot(h_ref[...], w_ref[...], preferred_element_type=_F32)
            + b_ref[...]
        )

    return pl.pallas_call(
        body,
        grid=(n // br,),
        in_specs=[
            pl.BlockSpec((br, D), lambda i: (i, 0)),
            pl.BlockSpec((D, D), lambda i: (0, 0)),
            pl.BlockSpec((1, D), lambda i: (0, 0)),
        ],
        out_specs=pl.BlockSpec((br, D), lambda i: (i, 0)),
        out_shape=jax.ShapeDtypeStruct((n, D), _F32),
    )(h, w, b.reshape(1, D))


def _pool_fc(psum, pcnt, s, bs, bg=2000):
    p3 = psum.reshape(2, GP, D)
    c3 = pcnt.reshape(2, GP, D)

    def body(p_ref, c_ref, s_ref, bs_ref, o_ref):
        cnt = jnp.maximum(c_ref[0, :, 0:1] + c_ref[1, :, 0:1], 1.0)
        pooled = (p_ref[0] + p_ref[1]) / cnt
        o_ref[...] = (
            jnp.dot(pooled, s_ref[...], preferred_element_type=_F32)
            + bs_ref[...]
        )

    return pl.pallas_call(
        body,
        grid=(G // bg,),
        in_specs=[
            pl.BlockSpec((2, bg, D), lambda i: (0, i, 0)),
            pl.BlockSpec((2, bg, D), lambda i: (0, i, 0)),
            pl.BlockSpec((D, D), lambda i: (0, 0)),
            pl.BlockSpec((1, D), lambda i: (0, 0)),
        ],
        out_specs=pl.BlockSpec((bg, D), lambda i: (i, 0)),
        out_shape=jax.ShapeDtypeStruct((G, D), _F32),
    )(p3, c3, s, bs.reshape(1, D))


def _final(psum, pcnt, fw1, fb1, fw2, fb2, bg=2000):
    p3 = psum.reshape(2, GP, D)
    c3 = pcnt.reshape(2, GP, D)
    nt = fw2.shape[1]

    def body(p_ref, c_ref, w1_ref, b1_ref, w2_ref, b2_ref, o_ref):
        cnt = jnp.maximum(c_ref[0, :, 0:1] + c_ref[1, :, 0:1], 1.0)
        hg = (p_ref[0] + p_ref[1]) / cnt
        t = jnp.maximum(
            jnp.dot(hg, w1_ref[...], preferred_element_type=_F32)
            + b1_ref[...],
            0.0,
        )
        o_ref[...] = (
            jnp.dot(t, w2_ref[...], preferred_element_type=_F32) + b2_ref[...]
        )

    return pl.pallas_call(
        body,
        grid=(G // bg,),
        in_specs=[
            pl.BlockSpec((2, bg, D), lambda i: (0, i, 0)),
            pl.BlockSpec((2, bg, D), lambda i: (0, i, 0)),
            pl.BlockSpec((D, 2 * D), lambda i: (0, 0)),
            pl.BlockSpec((1, 2 * D), lambda i: (0, 0)),
            pl.BlockSpec((2 * D, nt), lambda i: (0, 0)),
            pl.BlockSpec((1, nt), lambda i: (0, 0)),
        ],
        out_specs=pl.BlockSpec((bg, nt), lambda i: (i, 0)),
        out_shape=jax.ShapeDtypeStruct((G, nt), _F32),
    )(p3, c3, fw1, fb1.reshape(1, 2 * D), fw2, fb2.reshape(1, nt))


def kernel(h_subgraph, subgraph_idx_batch, W1, b1, W2, b2,
           S1, bs1, S2, bs2, Fw1, Fb1, Fw2, Fb2):
    n = h_subgraph.shape[0]
    assert n % (_NW * _CH) == 0
    ids = subgraph_idx_batch.astype(jnp.int32)

    (pcnt,) = _make_sc_count(n)(ids)
    (psum1,) = _make_sc_pool(n)(h_subgraph, ids)
    x1 = _mm_bias(h_subgraph, W1, b1)
    x2 = _pool_fc(psum1, pcnt, S1, bs1)
    h1, psum2 = _make_sc_update(n, True)(x1, x2, ids)
    x1b = _mm_bias(h1, W2, b2)
    x2b = _pool_fc(psum2, pcnt, S2, bs2)
    (psum3,) = _make_sc_update(n, False)(x1b, x2b, ids)
    return _final(psum3, pcnt, Fw1, Fb1, Fw2, Fb2)


# speedup vs baseline: 2.7166x; 1.4536x over previous
import functools

import jax
import jax.numpy as jnp
from jax import lax
from jax.experimental import pallas as pl
from jax.experimental.pallas import tpu as pltpu
from jax.experimental.pallas import tpu_sc as plsc

D = 128
G = 10000
GP = 10240
_NC = 2
_NS = 16
_NW = _NC * _NS
_CH = 80
_VPR = D // 16
_GS = GP // _NS

_F32 = jnp.float32


def _copy_ids(idall, c, idring, i4):
    for k in range(_CH // 16):
        idring[i4, pl.ds(16 * k, 16)] = idall[pl.ds(c * _CH + 16 * k, 16)]


def _sc_mesh():
    return plsc.VectorSubcoreMesh(core_axis_name="c", subcore_axis_name="s")


def _zero_fill(ref, rows):
    z = jnp.zeros((16,), _F32)

    def body(i, c):
        for j in range(_VPR):
            ref[i, pl.ds(j * 16, 16)] = z
        return c

    lax.fori_loop(0, rows, body, 0)


_ZR = 32


def _zero_shared(zbuf, acc_sh, sid):
    _zero_fill(zbuf, _ZR)
    for k in range(_GS // _ZR):
        pltpu.sync_copy(zbuf, acc_sh.at[pl.ds(sid * _GS + k * _ZR, _ZR)])
    plsc.subcore_barrier()


def _copy_out(acc_sh, out_hbm, cid, sid):
    plsc.subcore_barrier()
    out0 = cid * GP + sid * _GS
    pltpu.sync_copy(acc_sh.at[pl.ds(sid * _GS, _GS)],
                    out_hbm.at[pl.ds(out0, _GS)])


def _make_sc_pool(n):
    rw = n // _NW
    nch = rw // _CH
    kiters = nch // 4
    assert nch >= 8

    @functools.partial(
        pl.kernel,
        out_type=[jax.ShapeDtypeStruct((2 * GP, D), _F32)],
        mesh=_sc_mesh(),
        scratch_types=[
            pltpu.VMEM_SHARED((GP, D), _F32),
            [pltpu.VMEM((_CH, D), _F32) for _ in range(4)],
            [pltpu.VMEM((_CH,), jnp.int32) for _ in range(4)],
            pltpu.VMEM((_ZR, D), _F32),
            [pltpu.SemaphoreType.DMA for _ in range(4)],
            [pltpu.SemaphoreType.DMA for _ in range(4)],
            [pltpu.SemaphoreType.DMA for _ in range(4)],
        ],
    )
    def sc_pool(h_hbm, ids_hbm, psum_hbm, sum_sh, hb, idb, zbuf, sx, si, ss):
        cid = lax.axis_index("c")
        sid = lax.axis_index("s")
        wid = sid * _NC + cid
        base = wid * rw

        _zero_shared(zbuf, sum_sh, sid)

        def emit(c, i4):
            i4n = (i4 + 1) % 4

            @pl.when(c + 1 < nch)
            def _():
                @pl.when(c + 1 >= 4)
                def _():
                    pltpu.make_async_copy(
                        hb[i4n], sum_sh.at[idb[i4n]], ss[i4n]).wait()
                pltpu.async_copy(
                    ids_hbm.at[pl.ds(base + (c + 1) * _CH, _CH)],
                    idb[i4n], si[i4n])
                pltpu.async_copy(h_hbm.at[pl.ds(base + (c + 1) * _CH, _CH)],
                                 hb[i4n], sx[i4n])

            pltpu.make_async_copy(h_hbm.at[pl.ds(base + c * _CH, _CH)],
                                  hb[i4], sx[i4]).wait()
            pltpu.make_async_copy(ids_hbm.at[pl.ds(base + c * _CH, _CH)],
                                  idb[i4], si[i4]).wait()
            pltpu.async_copy(hb[i4], sum_sh.at[idb[i4]], ss[i4], add=True)

        pltpu.async_copy(ids_hbm.at[pl.ds(base, _CH)], idb[0], si[0])
        pltpu.async_copy(h_hbm.at[pl.ds(base, _CH)], hb[0], sx[0])

        def kbody(k, carry):
            for off in range(4):
                emit(4 * k + off, off)
            return carry

        lax.fori_loop(0, kiters, kbody, 0)
        for c in range(4 * kiters, nch):
            emit(c, c % 4)

        for c in range(nch - 4, nch):
            i4 = c % 4
            pltpu.make_async_copy(
                hb[i4], sum_sh.at[idb[i4]], ss[i4]).wait()

        _copy_out(sum_sh, psum_hbm, cid, sid)

    return sc_pool


def _make_sc_count(n):
    rw = n // _NW
    nch = rw // _CH
    kiters = nch // 4
    assert nch >= 8

    @functools.partial(
        pl.kernel,
        out_type=[jax.ShapeDtypeStruct((2 * GP, D), _F32)],
        mesh=_sc_mesh(),
        scratch_types=[
            pltpu.VMEM_SHARED((GP, D), _F32),
            pltpu.VMEM((_CH, D), _F32),
            [pltpu.VMEM((_CH,), jnp.int32) for _ in range(4)],
            pltpu.VMEM((_ZR, D), _F32),
            [pltpu.SemaphoreType.DMA for _ in range(4)],
            [pltpu.SemaphoreType.DMA for _ in range(4)],
        ],
    )
    def sc_count(ids_hbm, pcnt_hbm, cnt_sh, ones_v, idb, zbuf, si, ss):
        cid = lax.axis_index("c")
        sid = lax.axis_index("s")
        wid = sid * _NC + cid
        base = wid * rw

        _zero_shared(zbuf, cnt_sh, sid)
        one = jnp.full((16,), 1.0, _F32)

        def ofill(i, c):
            for j in range(_VPR):
                ones_v[i, pl.ds(j * 16, 16)] = one
            return c

        lax.fori_loop(0, _CH, ofill, 0)

        def emit(c, i4):
            i4n = (i4 + 1) % 4

            @pl.when(c + 1 < nch)
            def _():
                @pl.when(c + 1 >= 4)
                def _():
                    pltpu.make_async_copy(
                        ones_v, cnt_sh.at[idb[i4n]], ss[i4n]).wait()
                pltpu.async_copy(
                    ids_hbm.at[pl.ds(base + (c + 1) * _CH, _CH)],
                    idb[i4n], si[i4n])

            pltpu.make_async_copy(ids_hbm.at[pl.ds(base + c * _CH, _CH)],
                                  idb[i4], si[i4]).wait()
            pltpu.async_copy(ones_v, cnt_sh.at[idb[i4]], ss[i4], add=True)

        pltpu.async_copy(ids_hbm.at[pl.ds(base, _CH)], idb[0], si[0])

        def kbody(k, carry):
            for off in range(4):
                emit(4 * k + off, off)
            return carry

        lax.fori_loop(0, kiters, kbody, 0)
        for c in range(4 * kiters, nch):
            emit(c, c % 4)
        for c in range(nch - 4, nch):
            i4 = c % 4
            pltpu.make_async_copy(
                ones_v, cnt_sh.at[idb[i4]], ss[i4]).wait()

        _copy_out(cnt_sh, pcnt_hbm, cid, sid)

    return sc_count


def _make_sc_update(n):
    rw = n // _NW
    nch = rw // _CH
    kiters = nch // 4
    assert nch >= 8

    @functools.partial(
        pl.kernel,
        out_type=[jax.ShapeDtypeStruct((n, D), _F32)],
        mesh=_sc_mesh(),
        scratch_types=[
            [pltpu.VMEM((_CH, D), _F32) for _ in range(2)],
            [pltpu.VMEM((_CH, D), _F32) for _ in range(2)],
            [pltpu.VMEM((_CH, D), _F32) for _ in range(2)],
            [pltpu.VMEM((_CH,), jnp.int32) for _ in range(4)],
            [pltpu.SemaphoreType.DMA for _ in range(2)],
            [pltpu.SemaphoreType.DMA for _ in range(2)],
            [pltpu.SemaphoreType.DMA for _ in range(4)],
            [pltpu.SemaphoreType.DMA for _ in range(2)],
        ],
    )
    def sc_update(x1_hbm, x2_hbm, ids_hbm, hout_hbm,
                  xb, gb, ob, idb, sx, sg, si, so):
        cid = lax.axis_index("c")
        sid = lax.axis_index("s")
        wid = sid * _NC + cid
        base = wid * rw

        def emit(c, off):
            b = off % 2
            nb = 1 - b
            i4 = off % 4
            i4n = (off + 1) % 4
            i4nn = (off + 2) % 4

            pltpu.make_async_copy(x1_hbm.at[pl.ds(base + c * _CH, _CH)],
                                  xb[b], sx[b]).wait()
            pltpu.make_async_copy(x2_hbm.at[idb[i4]], gb[b], sg[b]).wait()

            @pl.when(c >= 2)
            def _():
                pltpu.make_async_copy(
                    ob[b],
                    hout_hbm.at[pl.ds(base + (c - 2) * _CH, _CH)],
                    so[b]).wait()

            @pl.when(c + 2 < nch)
            def _():
                pltpu.async_copy(
                    ids_hbm.at[pl.ds(base + (c + 2) * _CH, _CH)],
                    idb[i4nn], si[i4nn])

            @pl.when(c + 1 < nch)
            def _():
                pltpu.make_async_copy(
                    ids_hbm.at[pl.ds(base + (c + 1) * _CH, _CH)],
                    idb[i4n], si[i4n]).wait()
                pltpu.async_copy(
                    x1_hbm.at[pl.ds(base + (c + 1) * _CH, _CH)],
                    xb[nb], sx[nb])
                pltpu.async_copy(x2_hbm.at[idb[i4n]], gb[nb], sg[nb])

            @plsc.parallel_loop(0, _CH, unroll=4)
            def _(r):
                for j in range(_VPR):
                    sl = pl.ds(j * 16, 16)
                    v = xb[b][r, sl] + gb[b][r, sl]
                    ob[b][r, sl] = jnp.where(v > 0.0, v, jnp.exp(v) - 1.0)

            pltpu.async_copy(ob[b],
                             hout_hbm.at[pl.ds(base + c * _CH, _CH)],
                             so[b])

        pltpu.async_copy(ids_hbm.at[pl.ds(base, _CH)], idb[0], si[0])
        pltpu.async_copy(ids_hbm.at[pl.ds(base + _CH, _CH)], idb[1], si[1])
        pltpu.make_async_copy(ids_hbm.at[pl.ds(base, _CH)],
                              idb[0], si[0]).wait()
        pltpu.async_copy(x1_hbm.at[pl.ds(base, _CH)], xb[0], sx[0])
        pltpu.async_copy(x2_hbm.at[idb[0]], gb[0], sg[0])

        def kbody(k, carry):
            for off in range(4):
                emit(4 * k + off, off)
            return carry

        lax.fori_loop(0, kiters, kbody, 0)
        for c in range(4 * kiters, nch):
            emit(c, c % 4)

        for c in range(nch - 2, nch):
            b = c % 2
            pltpu.make_async_copy(
                ob[b], hout_hbm.at[pl.ds(base + c * _CH, _CH)],
                so[b]).wait()

    return sc_update


def _mm_bias(h, w, b, br=3200):
    n = h.shape[0]

    def body(h_ref, w_ref, b_ref, o_ref):
        o_ref[...] = (
            jnp.dot(h_ref[...], w_ref[...], preferred_element_type=_F32)
            + b_ref[...]
        )

    return pl.pallas_call(
        body,
        grid=(n // br,),
        in_specs=[
            pl.BlockSpec((br, D), lambda i: (i, 0)),
            pl.BlockSpec((D, D), lambda i: (0, 0)),
            pl.BlockSpec((1, D), lambda i: (0, 0)),
        ],
        out_specs=pl.BlockSpec((br, D), lambda i: (i, 0)),
        out_shape=jax.ShapeDtypeStruct((n, D), _F32),
    )(h, w, b.reshape(1, D))


def _pool_fc(psum, pcnt, s, bs, bg=2000):
    p3 = psum.reshape(2, GP, D)
    c3 = pcnt.reshape(2, GP, D)

    def body(p_ref, c_ref, s_ref, bs_ref, o_ref):
        cnt = jnp.maximum(c_ref[0, :, 0:1] + c_ref[1, :, 0:1], 1.0)
        pooled = (p_ref[0] + p_ref[1]) / cnt
        o_ref[...] = (
            jnp.dot(pooled, s_ref[...], preferred_element_type=_F32)
            + bs_ref[...]
        )

    return pl.pallas_call(
        body,
        grid=(G // bg,),
        in_specs=[
            pl.BlockSpec((2, bg, D), lambda i: (0, i, 0)),
            pl.BlockSpec((2, bg, D), lambda i: (0, i, 0)),
            pl.BlockSpec((D, D), lambda i: (0, 0)),
            pl.BlockSpec((1, D), lambda i: (0, 0)),
        ],
        out_specs=pl.BlockSpec((bg, D), lambda i: (i, 0)),
        out_shape=jax.ShapeDtypeStruct((G, D), _F32),
    )(p3, c3, s, bs.reshape(1, D))


def _final(psum, pcnt, fw1, fb1, fw2, fb2, bg=2000):
    p3 = psum.reshape(2, GP, D)
    c3 = pcnt.reshape(2, GP, D)
    nt = fw2.shape[1]

    def body(p_ref, c_ref, w1_ref, b1_ref, w2_ref, b2_ref, o_ref):
        cnt = jnp.maximum(c_ref[0, :, 0:1] + c_ref[1, :, 0:1], 1.0)
        hg = (p_ref[0] + p_ref[1]) / cnt
        t = jnp.maximum(
            jnp.dot(hg, w1_ref[...], preferred_element_type=_F32)
            + b1_ref[...],
            0.0,
        )
        o_ref[...] = (
            jnp.dot(t, w2_ref[...], preferred_element_type=_F32) + b2_ref[...]
        )

    return pl.pallas_call(
        body,
        grid=(G // bg,),
        in_specs=[
            pl.BlockSpec((2, bg, D), lambda i: (0, i, 0)),
            pl.BlockSpec((2, bg, D), lambda i: (0, i, 0)),
            pl.BlockSpec((D, 2 * D), lambda i: (0, 0)),
            pl.BlockSpec((1, 2 * D), lambda i: (0, 0)),
            pl.BlockSpec((2 * D, nt), lambda i: (0, 0)),
            pl.BlockSpec((1, nt), lambda i: (0, 0)),
        ],
        out_specs=pl.BlockSpec((bg, nt), lambda i: (i, 0)),
        out_shape=jax.ShapeDtypeStruct((G, nt), _F32),
    )(p3, c3, fw1, fb1.reshape(1, 2 * D), fw2, fb2.reshape(1, nt))


def kernel(h_subgraph, subgraph_idx_batch, W1, b1, W2, b2,
           S1, bs1, S2, bs2, Fw1, Fb1, Fw2, Fb2):
    n = h_subgraph.shape[0]
    assert n % (_NW * _CH) == 0
    ids = subgraph_idx_batch.astype(jnp.int32)

    (pcnt,) = _make_sc_count(n)(ids)
    (psum1,) = _make_sc_pool(n)(h_subgraph, ids)
    x1 = _mm_bias(h_subgraph, W1, b1)
    x2 = _pool_fc(psum1, pcnt, S1, bs1)
    (h1,) = _make_sc_update(n)(x1, x2, ids)
    (psum2,) = _make_sc_pool(n)(h1, ids)
    x1b = _mm_bias(h1, W2, b2)
    x2b = _pool_fc(psum2, pcnt, S2, bs2)
    (h2,) = _make_sc_update(n)(x1b, x2b, ids)
    (psum3,) = _make_sc_pool(n)(h2, ids)
    return _final(psum3, pcnt, Fw1, Fb1, Fw2, Fb2)
